# baseline (speedup 1.0000x reference)
import functools

import jax
import jax.numpy as jnp
from jax import lax
from jax.experimental import pallas as pl
from jax.experimental.pallas import tpu as pltpu
from jax.experimental.pallas import tpu_sc as plsc

N = 10000
E = 160000
D = 256
R = 32
H = 128
NPAD = 10240
NT = 16
EPT = 162 * 64
CH = 64
NG = 162
ZCH = 64
ZPT = 632
AGG_ROWS = NT * ZPT


def _mm_body(x_ref, w_ref, ws_ref, b_ref, t_ref, xwsb_ref):
    xb = x_ref[...]
    t_ref[...] = jnp.dot(xb, w_ref[...], preferred_element_type=jnp.float32)
    xwsb_ref[...] = (jnp.dot(xb, ws_ref[...],
                             preferred_element_type=jnp.float32)
                     + b_ref[...])[None]


_mm = pl.pallas_call(
    _mm_body,
    grid=(NPAD // 256, 2),
    in_specs=[
        pl.BlockSpec((256, 256), lambda i, h: (i, 0)),
        pl.BlockSpec((256, H), lambda i, h: (0, h)),
        pl.BlockSpec((256, H), lambda i, h: (0, h)),
        pl.BlockSpec((1, H), lambda i, h: (0, h)),
    ],
    out_specs=[
        pl.BlockSpec((256, H), lambda i, h: (h * (NPAD // 256) + i, 0)),
        pl.BlockSpec((1, 256, H), lambda i, h: (h, i, 0)),
    ],
    out_shape=[
        jax.ShapeDtypeStruct((2 * NPAD, H), jnp.float32),
        jax.ShapeDtypeStruct((2, NPAD, H), jnp.float32),
    ],
)


def _sc_body(t_hbm, idx_hbm, xwsb_hbm, out_hbm,
             idxbuf, idxg, rowbuf, relbuf, relw, agg,
             sem_i0, sem_i1, sem_i2, sem_i3, sem_i4, sem_i5,
             sem_r0, sem_r1, sem_r2, sem_e0, sem_e1,
             sem_s0, sem_s1, sem_s2):
    h = lax.axis_index("c")
    t = lax.axis_index("s")
    sem_i = (sem_i0, sem_i1, sem_i2, sem_i3, sem_i4, sem_i5)
    sem_r = (sem_r0, sem_r1, sem_r2)
    sem_e = (sem_e0, sem_e1)
    sem_s = (sem_s0, sem_s1, sem_s2)
    hoff = jnp.full((16,), h * NPAD, jnp.int32)

    @pl.when(t == 0)
    def _():
        pltpu.sync_copy(t_hbm.at[pl.ds(pl.multiple_of(h * NPAD + N, 8), R)],
                        relw)

    @plsc.parallel_loop(0, ZCH, 1, unroll=4)
    def _(i):
        for j in range(H // 16):
            rowbuf[0, i, pl.ds(j * 16, 16)] = jnp.zeros((16,), jnp.float32)

    zrows = (ZCH,) * 9 + (ZPT - 9 * ZCH,)
    for c in range(10):
        pltpu.sync_copy(rowbuf.at[0, pl.ds(0, zrows[c])],
                        agg.at[pl.ds(t * ZPT + c * ZCH, zrows[c])])
    plsc.subcore_barrier()

    def idx_start(g, s):
        pltpu.async_copy(idx_hbm.at[t, g], idxbuf.at[s], sem_i[s])

    def idx_wait(s):
        pltpu.make_async_copy(idx_hbm.at[t, 0], idxbuf.at[s],
                              sem_i[s]).wait()

    def gather_start(b, s, rb):
        for j in range(CH // 16):
            sl = pl.ds(j * 16, 16)
            idxg[b, 0, sl] = idxbuf[s, 0, sl] + hoff
        pltpu.async_copy(t_hbm.at[idxg.at[b, 0]], rowbuf.at[b], sem_r[b])
        pltpu.async_copy(relw.at[idxbuf.at[s, 1]], relbuf.at[rb], sem_e[rb])

    def gather_wait(b, s, rb):
        pltpu.make_async_copy(t_hbm.at[idxg.at[b, 0]], rowbuf.at[b],
                              sem_r[b]).wait()
        pltpu.make_async_copy(relw.at[idxbuf.at[s, 1]], relbuf.at[rb],
                              sem_e[rb]).wait()

    def scatter_wait(b, s):
        pltpu.make_async_copy(rowbuf.at[b], agg.at[idxbuf.at[s, 2]],
                              sem_s[b]).wait()

    idx_start(0, 0)
    idx_start(1, 1)
    idx_start(2, 2)
    idx_wait(0)
    gather_start(0, 0, 0)
    idx_wait(1)
    gather_start(1, 1, 1)

    def half_step(g, b, si, rb):
        b2 = (si + 2) % 3
        si2 = (si + 2) % 6
        pb = (si + 2) % 3

        @pl.when(g + 3 < NG)
        def _():
            idx_start(g + 3, (si + 3) % 6)

        gather_wait(b, si, rb)

        @plsc.parallel_loop(0, CH, 1, unroll=4)
        def _(i):
            for j in range(H // 16):
                sl = pl.ds(j * 16, 16)
                rowbuf[b, i, sl] = jnp.maximum(
                    rowbuf[b, i, sl] + relbuf[b, i, sl], 0.0)

        @pl.when(g >= 1)
        def _():
            scatter_wait(pb, (si + 5) % 6)

        @pl.when(g + 2 < NG)
        def _():
            idx_wait(si2)
            gather_start(b2, si2, si % 2)

        pltpu.async_copy(rowbuf.at[b], agg.at[idxbuf.at[si, 2]], sem_s[b],
                         add=True)

    @pl.loop(0, NG // 6)
    def _(gq):
        for r in range(6):
            half_step(gq * 6 + r, r % 3, r, r % 2)

    scatter_wait((NG - 1) % 3, (NG - 1) % 6)
    plsc.subcore_barrier()

    for c in range(10):
        row0 = jnp.minimum(t * ZPT + c * ZCH, N - ZCH)
        row0 = pl.multiple_of(row0, 8)
        pltpu.sync_copy(agg.at[pl.ds(row0, ZCH)], rowbuf.at[0, pl.ds(0, ZCH)])
        pltpu.sync_copy(xwsb_hbm.at[h, pl.ds(row0, ZCH)],
                        rowbuf.at[1, pl.ds(0, ZCH)])

        @plsc.parallel_loop(0, ZCH, 1, unroll=4)
        def _(r):
            for j in range(H // 16):
                sl = pl.ds(j * 16, 16)
                rowbuf[0, r, sl] = jnp.maximum(
                    rowbuf[0, r, sl] + rowbuf[1, r, sl], 0.0)

        pltpu.sync_copy(rowbuf.at[0, pl.ds(0, ZCH)],
                        out_hbm.at[pl.ds(row0, ZCH), pl.ds(h * H, H)])


_sc_edge = functools.partial(
    pl.kernel,
    out_type=jax.ShapeDtypeStruct((N, D), jnp.float32),
    mesh=plsc.VectorSubcoreMesh(core_axis_name="c", subcore_axis_name="s",
                                num_cores=2, num_subcores=NT),
    scratch_types=[
        pltpu.VMEM((6, 3, CH), jnp.int32),
        pltpu.VMEM((3, 1, CH), jnp.int32),
        pltpu.VMEM((3, CH, H), jnp.float32),
        pltpu.VMEM((2, CH, H), jnp.float32),
        pltpu.VMEM_SHARED((R, H), jnp.float32),
        pltpu.VMEM_SHARED((AGG_ROWS, H), jnp.float32),
    ] + [pltpu.SemaphoreType.DMA] * 14,
)(_sc_body)


def kernel(x, edge_index, edge_type, W, W_self, rel_emb, bias):
    src = edge_index[0]
    dst = edge_index[1]
    et = edge_type

    xc = jnp.concatenate(
        [x, rel_emb, jnp.zeros((NPAD - N - R, D), jnp.float32)], axis=0)
    bias2 = bias.reshape(1, D)
    t_tab, xwsb = _mm(xc, W, W_self, bias2)

    pad = EPT - E // NT
    srcp = jnp.pad(src.reshape(NT, E // NT), ((0, 0), (0, pad)))
    etp = jnp.pad(et.reshape(NT, E // NT), ((0, 0), (0, pad)))
    dstp = jnp.pad(dst.reshape(NT, E // NT), ((0, 0), (0, pad)),
                   constant_values=N)
    srcc = srcp.reshape(NT, NG, CH)
    relc = etp.reshape(NT, NG, CH)
    dstc = dstp.reshape(NT, NG, CH)
    idx_all = jnp.stack([srcc, relc, dstc], axis=2)

    return _sc_edge(t_tab, idx_all, xwsb)


# speedup vs baseline: 1.1043x; 1.1043x over previous
import functools

import jax
import jax.numpy as jnp
from jax import lax
from jax.experimental import pallas as pl
from jax.experimental.pallas import tpu as pltpu
from jax.experimental.pallas import tpu_sc as plsc

N = 10000
E = 160000
D = 256
R = 32
H = 128
NPAD = 10240
NT = 16
EPT = NPAD
CH = 80
NG = EPT // CH
ZCH = 64
ZPT = 632
AGG_ROWS = NT * ZPT


def _mm_body(x_ref, w_ref, ws_ref, b_ref, t_ref, xwsb_ref):
    xb = x_ref[...]
    t_ref[...] = jnp.dot(xb, w_ref[...], preferred_element_type=jnp.float32)
    xwsb_ref[...] = (jnp.dot(xb, ws_ref[...],
                             preferred_element_type=jnp.float32)
                     + b_ref[...])[None]


_mm = pl.pallas_call(
    _mm_body,
    grid=(NPAD // 256, 2),
    in_specs=[
        pl.BlockSpec((256, 256), lambda i, h: (i, 0)),
        pl.BlockSpec((256, H), lambda i, h: (0, h)),
        pl.BlockSpec((256, H), lambda i, h: (0, h)),
        pl.BlockSpec((1, H), lambda i, h: (0, h)),
    ],
    out_specs=[
        pl.BlockSpec((256, H), lambda i, h: (h * (NPAD // 256) + i, 0)),
        pl.BlockSpec((1, 256, H), lambda i, h: (h, i, 0)),
    ],
    out_shape=[
        jax.ShapeDtypeStruct((2 * NPAD, H), jnp.float32),
        jax.ShapeDtypeStruct((2, NPAD, H), jnp.float32),
    ],
)


def _sc_body(t_hbm, idx_hbm, xwsb_hbm, out_hbm,
             idxbuf, idxg, rowbuf, relbuf, relw, agg,
             sem_i0, sem_i1, sem_i2, sem_i3,
             sem_r0, sem_r1, sem_e0, sem_e1, sem_s0, sem_s1):
    h = lax.axis_index("c")
    t = lax.axis_index("s")
    sem_i = (sem_i0, sem_i1, sem_i2, sem_i3)
    sem_r = (sem_r0, sem_r1)
    sem_e = (sem_e0, sem_e1)
    sem_s = (sem_s0, sem_s1)
    hoff = jnp.full((16,), h * NPAD, jnp.int32)

    @pl.when(t == 0)
    def _():
        pltpu.sync_copy(t_hbm.at[pl.ds(pl.multiple_of(h * NPAD + N, 8), R)],
                        relw)

    @plsc.parallel_loop(0, ZCH, 1, unroll=4)
    def _(i):
        for j in range(H // 16):
            rowbuf[0, i, pl.ds(j * 16, 16)] = jnp.zeros((16,), jnp.float32)

    zrows = (ZCH,) * 9 + (ZPT - 9 * ZCH,)
    for c in range(10):
        pltpu.sync_copy(rowbuf.at[0, pl.ds(0, zrows[c])],
                        agg.at[pl.ds(t * ZPT + c * ZCH, zrows[c])])
    plsc.subcore_barrier()

    def idx_start(g, s):
        pltpu.async_copy(idx_hbm.at[t, g], idxbuf.at[s], sem_i[s])

    def idx_wait(s):
        pltpu.make_async_copy(idx_hbm.at[t, 0], idxbuf.at[s],
                              sem_i[s]).wait()

    def gather_start(b, s):
        for j in range(CH // 16):
            sl = pl.ds(j * 16, 16)
            idxg[b, 0, sl] = idxbuf[s, 0, sl] + hoff
        pltpu.async_copy(t_hbm.at[idxg.at[b, 0]], rowbuf.at[b], sem_r[b])
        pltpu.async_copy(relw.at[idxbuf.at[s, 1]], relbuf.at[b], sem_e[b])

    def gather_wait(b, s):
        pltpu.make_async_copy(t_hbm.at[idxg.at[b, 0]], rowbuf.at[b],
                              sem_r[b]).wait()
        pltpu.make_async_copy(relw.at[idxbuf.at[s, 1]], relbuf.at[b],
                              sem_e[b]).wait()

    def scatter_wait(b, s):
        pltpu.make_async_copy(rowbuf.at[b], agg.at[idxbuf.at[s, 2]],
                              sem_s[b]).wait()

    idx_start(0, 0)
    idx_start(1, 1)
    idx_start(2, 2)
    idx_wait(0)
    gather_start(0, 0)

    def half_step(g, b, bi):
        nb = 1 - b
        nbi = (bi + 1) % 4
        pbi = (bi + 3) % 4

        @pl.when(g >= 1)
        def _():
            scatter_wait(nb, pbi)

        @pl.when(g + 3 < NG)
        def _():
            idx_start(g + 3, pbi)

        @pl.when(g + 1 < NG)
        def _():
            idx_wait(nbi)
            gather_start(nb, nbi)

        gather_wait(b, bi)

        @plsc.parallel_loop(0, CH, 1, unroll=4)
        def _(i):
            for j in range(H // 16):
                sl = pl.ds(j * 16, 16)
                rowbuf[b, i, sl] = jnp.maximum(
                    rowbuf[b, i, sl] + relbuf[b, i, sl], 0.0)

        pltpu.async_copy(rowbuf.at[b], agg.at[idxbuf.at[bi, 2]], sem_s[b],
                         add=True)

    @pl.loop(0, NG // 4)
    def _(gq):
        half_step(gq * 4, 0, 0)
        half_step(gq * 4 + 1, 1, 1)
        half_step(gq * 4 + 2, 0, 2)
        half_step(gq * 4 + 3, 1, 3)

    scatter_wait((NG - 1) % 2, (NG - 1) % 4)
    plsc.subcore_barrier()

    for c in range(10):
        row0 = jnp.minimum(t * ZPT + c * ZCH, N - ZCH)
        row0 = pl.multiple_of(row0, 8)
        pltpu.sync_copy(agg.at[pl.ds(row0, ZCH)], rowbuf.at[0, pl.ds(0, ZCH)])
        pltpu.sync_copy(xwsb_hbm.at[h, pl.ds(row0, ZCH)],
                        rowbuf.at[1, pl.ds(0, ZCH)])

        @plsc.parallel_loop(0, ZCH, 1, unroll=4)
        def _(r):
            for j in range(H // 16):
                sl = pl.ds(j * 16, 16)
                rowbuf[0, r, sl] = jnp.maximum(
                    rowbuf[0, r, sl] + rowbuf[1, r, sl], 0.0)

        pltpu.sync_copy(rowbuf.at[0, pl.ds(0, ZCH)],
                        out_hbm.at[pl.ds(row0, ZCH), pl.ds(h * H, H)])


_sc_edge = functools.partial(
    pl.kernel,
    out_type=jax.ShapeDtypeStruct((N, D), jnp.float32),
    mesh=plsc.VectorSubcoreMesh(core_axis_name="c", subcore_axis_name="s",
                                num_cores=2, num_subcores=NT),
    scratch_types=[
        pltpu.VMEM((4, 3, CH), jnp.int32),
        pltpu.VMEM((2, 2, CH), jnp.int32),
        pltpu.VMEM((2, CH, H), jnp.float32),
        pltpu.VMEM((2, CH, H), jnp.float32),
        pltpu.VMEM_SHARED((R, H), jnp.float32),
        pltpu.VMEM_SHARED((AGG_ROWS, H), jnp.float32),
        pltpu.SemaphoreType.DMA,
        pltpu.SemaphoreType.DMA,
        pltpu.SemaphoreType.DMA,
        pltpu.SemaphoreType.DMA,
        pltpu.SemaphoreType.DMA,
        pltpu.SemaphoreType.DMA,
        pltpu.SemaphoreType.DMA,
        pltpu.SemaphoreType.DMA,
        pltpu.SemaphoreType.DMA,
        pltpu.SemaphoreType.DMA,
    ],
)(_sc_body)


def kernel(x, edge_index, edge_type, W, W_self, rel_emb, bias):
    src = edge_index[0]
    dst = edge_index[1]
    et = edge_type

    xc = jnp.concatenate(
        [x, rel_emb, jnp.zeros((NPAD - N - R, D), jnp.float32)], axis=0)
    bias2 = bias.reshape(1, D)
    t_tab, xwsb = _mm(xc, W, W_self, bias2)

    pad = EPT - E // NT
    srcp = jnp.pad(src.reshape(NT, E // NT), ((0, 0), (0, pad)))
    etp = jnp.pad(et.reshape(NT, E // NT), ((0, 0), (0, pad)))
    dstp = jnp.pad(dst.reshape(NT, E // NT), ((0, 0), (0, pad)),
                   constant_values=N)
    srcc = srcp.reshape(NT, NG, CH)
    relc = etp.reshape(NT, NG, CH)
    dstc = dstp.reshape(NT, NG, CH)
    idx_all = jnp.stack([srcc, relc, dstc], axis=2)

    return _sc_edge(t_tab, idx_all, xwsb)
